# Initial kernel scaffold; baseline (speedup 1.0000x reference)
import jax
import jax.numpy as jnp
from jax.experimental import pallas as pl


def kernel(points, features, leaf_mask, W1, b1, W2, b2, W3, b3):
    raise NotImplementedError("write your pallas kernel here")


# speedup vs baseline: 4.8738x; 4.8738x over previous
import functools

import jax
import jax.numpy as jnp
from jax.experimental import pallas as pl
from jax.experimental.pallas import tpu as pltpu

B, N, FD = 4, 4096, 64
BN = B * N
_LOG2N = 12


def _shift_right(x, s):
    return jnp.concatenate(
        [jnp.zeros(x.shape[:-1] + (s,), x.dtype), x[..., : x.shape[-1] - s]],
        axis=-1,
    )


def _body(fT_ref, pT_ref, m_ref, W1f_ref, W1p_ref, b1_ref, W2_ref, b2_ref,
          W3_ref, b3_ref, prob_ref, conf_ref):
    fT = fT_ref[...]
    pT = pT_ref[...]
    m = m_ref[...]

    h1 = jnp.dot(W1f_ref[...], fT, preferred_element_type=jnp.float32)
    h1 += jnp.dot(W1p_ref[...], pT, preferred_element_type=jnp.float32)
    h1 = jnp.maximum(h1 + b1_ref[...], 0.0)
    h2 = jnp.maximum(
        jnp.dot(W2_ref[...], h1, preferred_element_type=jnp.float32)
        + b2_ref[...], 0.0)
    logit = (jnp.dot(W3_ref[...], h2, preferred_element_type=jnp.float32)
             + b3_ref[...])
    srow = jax.nn.sigmoid(logit)

    lane128 = jax.lax.broadcasted_iota(jnp.int32, (1, 128), 1)
    conf_vec = jnp.zeros((1, 128), jnp.float32)

    for b in range(B):
        cols = slice(b * N, (b + 1) * N)
        m_b = m[:, cols]
        cnt = jnp.sum(m_b)
        prob = jnp.where(m_b > 0.5, srow[:, cols], 0.0)
        prob = jnp.where(cnt < 10.0, 0.0, prob)
        prob_ref[:, cols] = prob

        mean = jnp.sum(prob * m_b) / jnp.maximum(cnt, 1.0)
        clarity = (jnp.sum(m_b * (prob - mean) ** 2)
                   / jnp.maximum(cnt - 1.0, 1.0))

        sel = (prob > 0.7).astype(jnp.float32)
        bcnt = jnp.sum(sel)
        P = pT[:, cols]
        has = sel
        val = P * sel
        for k in range(_LOG2N):
            s = 1 << k
            has_s = _shift_right(has, s)
            val_s = _shift_right(val, s)
            val = jnp.where(has > 0.5, val, val_s)
            has = jnp.maximum(has, has_s)
        ffprev = _shift_right(val, 1)
        hasprev = _shift_right(has, 1)
        valid = sel * hasprev
        delta = P - ffprev
        dsq = jnp.sum(delta * delta, axis=0, keepdims=True)
        dist = jnp.sqrt(jnp.maximum(dsq, 1e-24))
        sum_d = jnp.sum(valid * dist)
        pc = jnp.maximum(bcnt - 1.0, 1.0)
        dmean = sum_d / pc
        dvar = jnp.sum(valid * (dist - dmean) ** 2) / jnp.maximum(pc - 1.0, 1.0)
        continuity = jnp.clip(1.0 / (dvar + 1e-8), 0.0, 1.0)
        continuity = jnp.where(bcnt > 5.0, continuity, 0.0)
        conf = jnp.clip(clarity * continuity, 0.0, 1.0)
        conf = jnp.where(cnt == 0.0, 0.0, conf)
        conf_vec += jnp.where(lane128 == b, conf, 0.0)

    conf_ref[...] = jnp.broadcast_to(conf_vec, (8, 128))


@functools.partial(jax.jit, static_argnames=())
def kernel(points, features, leaf_mask, W1, b1, W2, b2, W3, b3):
    fT = jnp.transpose(features, (2, 0, 1)).reshape(FD, BN)
    pT = jnp.transpose(points, (2, 0, 1)).reshape(3, BN)
    mrow = leaf_mask.astype(jnp.float32).reshape(1, BN)
    W1f = W1[:, :FD]
    W1p = W1[:, FD:]
    b1c = b1.reshape(FD, 1)
    b2c = b2.reshape(32, 1)
    b3c = b3.reshape(1, 1)

    prob_row, conf_pad = pl.pallas_call(
        _body,
        out_shape=(
            jax.ShapeDtypeStruct((1, BN), jnp.float32),
            jax.ShapeDtypeStruct((8, 128), jnp.float32),
        ),
    )(fT, pT, mrow, W1f, W1p, b1c, W2, b2c, W3, b3c)

    boundary_prob = prob_row.reshape(B, N)
    separation_confidence = conf_pad[0, :B]
    return (boundary_prob, features, separation_confidence)
